# baseline (speedup 1.0000x reference)
import functools

import jax
import jax.numpy as jnp
from jax import lax
from jax.experimental import pallas as pl
from jax.experimental.pallas import tpu as pltpu
from jax.experimental.pallas import tpu_sc as plsc

_EMBED = 1024
_E = 16
_B = 128

_NW = 32
_ROWS = _E * _EMBED
_RPW = _ROWS // _NW
_CHUNK = 32
_NCHUNK = _RPW // _CHUNK
_LANES = 16


def _rowsum_sc(W_flat):
    mesh = plsc.VectorSubcoreMesh(core_axis_name="c", subcore_axis_name="s")

    @functools.partial(
        pl.kernel,
        mesh=mesh,
        out_type=jax.ShapeDtypeStruct((_ROWS,), jnp.float32),
        scratch_types=[
            pltpu.VMEM((_CHUNK * _EMBED,), jnp.float32),
            pltpu.VMEM((_CHUNK * _EMBED,), jnp.float32),
            pltpu.VMEM((_RPW,), jnp.float32),
            pltpu.VMEM((_CHUNK * _LANES,), jnp.float32),
            pltpu.SemaphoreType.DMA,
            pltpu.SemaphoreType.DMA,
        ],
        compiler_params=pltpu.CompilerParams(needs_layout_passes=False),
    )
    def k(w_hbm, out_hbm, buf0, buf1, res, rowpart, sem0, sem1):
        wid = lax.axis_index("s") * 2 + lax.axis_index("c")
        base = wid * _RPW * _EMBED
        bufs = (buf0, buf1)
        sems = (sem0, sem1)

        def start(c):
            off = base + c * _CHUNK * _EMBED
            return pltpu.async_copy(
                w_hbm.at[pl.ds(off, _CHUNK * _EMBED)], bufs[c % 2], sems[c % 2])

        lane = lax.iota(jnp.int32, _LANES)

        def shuf(v, idx):
            return v.at[idx].get(mode="promise_in_bounds",
                                 unique_indices=True)

        def combine(a, b, d):
            mask = (lane & d) == 0
            return (jnp.where(mask, a, shuf(b, lane ^ d))
                    + jnp.where(mask, shuf(a, lane ^ d), b))

        bitrev = (((lane & 1) << 3) | ((lane & 2) << 1)
                  | ((lane & 4) >> 1) | ((lane & 8) >> 3))

        cp = start(0)
        for c in range(_NCHUNK):
            nxt = start(c + 1) if c + 1 < _NCHUNK else None
            cp.wait()
            buf = bufs[c % 2]

            @plsc.parallel_loop(0, _CHUNK, unroll=2)
            def _row(r, buf=buf):
                off = r * _EMBED
                accs = [jnp.zeros((_LANES,), jnp.float32) for _ in range(4)]
                for k in range(_EMBED // _LANES):
                    accs[k % 4] = accs[k % 4] + buf[pl.ds(off + k * _LANES,
                                                          _LANES)]
                rowpart[pl.ds(r * _LANES, _LANES)] = (
                    (accs[0] + accs[1]) + (accs[2] + accs[3]))

            for g in range(_CHUNK // _LANES):
                cur = [rowpart[pl.ds((g * _LANES + i) * _LANES, _LANES)]
                       for i in range(_LANES)]
                for d in (8, 4, 2, 1):
                    cur = [combine(cur[2 * j], cur[2 * j + 1], d)
                           for j in range(len(cur) // 2)]
                res[pl.ds(c * _CHUNK + g * _LANES, _LANES)] = shuf(
                    cur[0], bitrev)
            cp = nxt
        pltpu.sync_copy(res, out_hbm.at[pl.ds(wid * _RPW, _RPW)])

    return k(W_flat)


def _combine_kernel(x_ref, Wg_ref, bg_ref, ws_ref, b_ref, out_ref):
    logits = x_ref[...] @ Wg_ref[...] + bg_ref[...]
    m = jnp.max(logits, axis=1, keepdims=True)
    p = jnp.exp(logits - m)
    g = 1.0 / jnp.sum(p, axis=1)
    ii = jax.lax.broadcasted_iota(jnp.int32, (_B, _E), 1)
    idx = jnp.min(jnp.where(logits == m, ii, _E), axis=1)
    S = lax.dot_general(x_ref[...], ws_ref[...],
                        (((1,), (1,)), ((), ())))
    bsum = jnp.sum(b_ref[...], axis=1)
    A = g[:, None] * (S + bsum[None, :])
    H = (ii == idx[:, None]).astype(jnp.float32)
    out_ref[...] = A @ H.T


def kernel(x, Wg, bg, W, b):
    w_sum = _rowsum_sc(W.reshape(_ROWS * _EMBED)).reshape(_E, _EMBED)
    return pl.pallas_call(
        _combine_kernel,
        out_shape=jax.ShapeDtypeStruct((_B, _B), jnp.float32),
    )(x, Wg, bg.reshape(1, _E), w_sum, b)


# speedup vs baseline: 1.9717x; 1.9717x over previous
import functools

import jax
import jax.numpy as jnp
from jax import lax
from jax.experimental import pallas as pl
from jax.experimental.pallas import tpu as pltpu
from jax.experimental.pallas import tpu_sc as plsc

_EMBED = 1024
_E = 16
_B = 128

_NW = 32
_ROWS = _E * _EMBED
_RPW = _ROWS // _NW
_CHUNK = 32
_RING = 3
_NCHUNK = _RPW // _CHUNK
_LANES = 16


def _rowsum_sc(W_flat):
    mesh = plsc.VectorSubcoreMesh(core_axis_name="c", subcore_axis_name="s")

    @functools.partial(
        pl.kernel,
        mesh=mesh,
        out_type=jax.ShapeDtypeStruct((_ROWS,), jnp.float32),
        scratch_types=[
            [pltpu.VMEM((_CHUNK, _EMBED), jnp.float32)] * _RING,
            pltpu.VMEM((_RPW,), jnp.float32),
            pltpu.VMEM((_CHUNK * _LANES,), jnp.float32),
            [pltpu.SemaphoreType.DMA] * _RING,
        ],
        compiler_params=pltpu.CompilerParams(needs_layout_passes=False),
    )
    def k(w_hbm, out_hbm, bufs, res, rowpart, sems):
        wid = lax.axis_index("s") * 2 + lax.axis_index("c")
        base = wid * _RPW * _EMBED
        def start(c):
            row0 = wid * _RPW + c * _CHUNK
            return pltpu.async_copy(
                w_hbm.at[pl.ds(row0, _CHUNK)], bufs[c % _RING],
                sems[c % _RING])

        lane = lax.iota(jnp.int32, _LANES)

        def shuf(v, idx):
            return v.at[idx].get(mode="promise_in_bounds",
                                 unique_indices=True)

        def combine(a, b, d):
            mask = (lane & d) == 0
            return (jnp.where(mask, a, shuf(b, lane ^ d))
                    + jnp.where(mask, shuf(a, lane ^ d), b))

        bitrev = (((lane & 1) << 3) | ((lane & 2) << 1)
                  | ((lane & 4) >> 1) | ((lane & 8) >> 3))

        cps = [start(c) for c in range(_RING)]
        for c in range(_NCHUNK):
            cps[c % _RING].wait()
            buf = bufs[c % _RING]
            @plsc.parallel_loop(0, _CHUNK, unroll=2)
            def _row(r, buf=buf):
                accs = [jnp.zeros((_LANES,), jnp.float32) for _ in range(4)]
                for k in range(_EMBED // _LANES):
                    accs[k % 4] = accs[k % 4] + buf[r, pl.ds(k * _LANES,
                                                             _LANES)]
                rowpart[pl.ds(r * _LANES, _LANES)] = (
                    (accs[0] + accs[1]) + (accs[2] + accs[3]))

            for g in range(_CHUNK // _LANES):
                cur = [rowpart[pl.ds((g * _LANES + i) * _LANES, _LANES)]
                       for i in range(_LANES)]
                for d in (8, 4, 2, 1):
                    cur = [combine(cur[2 * j], cur[2 * j + 1], d)
                           for j in range(len(cur) // 2)]
                res[pl.ds(c * _CHUNK + g * _LANES, _LANES)] = shuf(
                    cur[0], bitrev)
            if c + _RING < _NCHUNK:
                cps[c % _RING] = start(c + _RING)
        pltpu.sync_copy(res, out_hbm.at[pl.ds(wid * _RPW, _RPW)])

    return k(W_flat.reshape(_ROWS, _EMBED))


def _combine_kernel(x_ref, Wg_ref, bg_ref, ws_ref, b_ref, out_ref):
    logits = x_ref[...] @ Wg_ref[...] + bg_ref[...]
    m = jnp.max(logits, axis=1, keepdims=True)
    p = jnp.exp(logits - m)
    g = 1.0 / jnp.sum(p, axis=1)
    ii = jax.lax.broadcasted_iota(jnp.int32, (_B, _E), 1)
    idx = jnp.min(jnp.where(logits == m, ii, _E), axis=1)
    S = lax.dot_general(x_ref[...], ws_ref[...],
                        (((1,), (1,)), ((), ())))
    bsum = jnp.sum(b_ref[...], axis=1)
    A = g[:, None] * (S + bsum[None, :])
    H = (ii == idx[:, None]).astype(jnp.float32)
    out_ref[...] = A @ H.T


def kernel(x, Wg, bg, W, b):
    w_sum = _rowsum_sc(W.reshape(_ROWS * _EMBED)).reshape(_E, _EMBED)
    return pl.pallas_call(
        _combine_kernel,
        out_shape=jax.ShapeDtypeStruct((_B, _B), jnp.float32),
    )(x, Wg, bg.reshape(1, _E), w_sum, b)
